# baseline (speedup 1.0000x reference)
import jax
import jax.numpy as jnp
from jax import lax
from jax.experimental import pallas as pl
from jax.experimental.pallas import tpu as pltpu
from jax.experimental.pallas import tpu_sc as plsc

B, K_SLOTS, N, DIM, NUM_ENTRIES = 16, 64, 1024, 768, 8192

_NC, _NS = 2, 16
_NW = _NC * _NS
FB = 8
SB = B - FB
BB = 4
_W_PER_B = _NW // SB
_K_PER_W = K_SLOTS // _W_PER_B


def _sc_gather_body(idx_hbm, table_hbm, out_hbm, idx_v, rows_v, sem):
    wid = lax.axis_index("s") * _NC + lax.axis_index("c")
    b = wid // _W_PER_B
    col = (wid % _W_PER_B) * _K_PER_W
    pltpu.sync_copy(idx_hbm.at[FB + b, pl.ds(col, _K_PER_W)], idx_v)
    pltpu.async_copy(table_hbm.at[idx_v], rows_v, sem).wait()
    pltpu.sync_copy(rows_v, out_hbm.at[b, pl.ds(col, _K_PER_W)])


_sc_gather_half = pl.kernel(
    _sc_gather_body,
    out_type=jax.ShapeDtypeStruct((SB, K_SLOTS, DIM), jnp.float32),
    mesh=plsc.VectorSubcoreMesh(core_axis_name="c", subcore_axis_name="s"),
    scratch_types=[
        pltpu.VMEM((_K_PER_W,), jnp.int32),
        pltpu.VMEM((_K_PER_W, DIM), jnp.float32),
        pltpu.SemaphoreType.DMA,
    ],
)

_MM_DIMS = (((1,), (1,)), ((0,), (0,)))


def _mm_fused_body(idx_ref, attn_ref, table_ref, out_ref, rows_scr, sems):
    g = pl.program_id(0)
    n_steps = FB // BB

    def issue(gi, slot):
        for j in range(BB):
            for k in range(K_SLOTS):
                pltpu.make_async_copy(
                    table_ref.at[idx_ref[gi * BB + j, k]],
                    rows_scr.at[slot, j, k],
                    sems.at[slot],
                ).start()

    def drain(gi, slot):
        for j in range(BB):
            for k in range(K_SLOTS):
                pltpu.make_async_copy(
                    table_ref.at[idx_ref[gi * BB + j, k]],
                    rows_scr.at[slot, j, k],
                    sems.at[slot],
                ).wait()

    @pl.when(g == 0)
    def _():
        issue(0, 0)

    @pl.when(g + 1 < n_steps)
    def _():
        issue(g + 1, (g + 1) % 2)

    drain(g, g % 2)
    out_ref[...] = lax.dot_general(
        attn_ref[...],
        rows_scr[g % 2],
        _MM_DIMS,
        preferred_element_type=jnp.float32,
    )


def _mm_body1(attn_ref, rows_ref, prev_ref, out_ref):
    del prev_ref
    out_ref[...] = lax.dot_general(
        attn_ref[...],
        rows_ref[...],
        _MM_DIMS,
        preferred_element_type=jnp.float32,
    )


@jax.jit
def kernel(indices, attn_map, prompt):
    rows1 = _sc_gather_half(indices, prompt)
    out0 = pl.pallas_call(
        _mm_fused_body,
        grid=(FB // BB,),
        in_specs=[
            pl.BlockSpec(memory_space=pltpu.SMEM),
            pl.BlockSpec((BB, K_SLOTS, N), lambda g: (g, 0, 0)),
            pl.BlockSpec(memory_space=pl.ANY),
        ],
        out_specs=pl.BlockSpec((BB, N, DIM), lambda g: (g, 0, 0)),
        out_shape=jax.ShapeDtypeStruct((B, N, DIM), jnp.float32),
        scratch_shapes=[
            pltpu.VMEM((2, BB, K_SLOTS, DIM), jnp.float32),
            pltpu.SemaphoreType.DMA((2,)),
        ],
    )(indices, attn_map, prompt)
    out = pl.pallas_call(
        _mm_body1,
        grid=(SB // BB,),
        in_specs=[
            pl.BlockSpec((BB, K_SLOTS, N), lambda g: (g + FB // BB, 0, 0)),
            pl.BlockSpec((BB, K_SLOTS, DIM), lambda g: (g, 0, 0)),
            pl.BlockSpec(memory_space=pl.ANY),
        ],
        out_specs=pl.BlockSpec((BB, N, DIM), lambda g: (g + FB // BB, 0, 0)),
        out_shape=jax.ShapeDtypeStruct((B, N, DIM), jnp.float32),
        input_output_aliases={2: 0},
    )(attn_map, rows1, out0)
    return out


# speedup vs baseline: 1.0374x; 1.0374x over previous
import jax
import jax.numpy as jnp
from jax import lax
from jax.experimental import pallas as pl
from jax.experimental.pallas import tpu as pltpu
from jax.experimental.pallas import tpu_sc as plsc

B, K_SLOTS, N, DIM, NUM_ENTRIES = 16, 64, 1024, 768, 8192

_NC, _NS = 2, 16
_NW = _NC * _NS
FB = 8
SB = B - FB
BB = 2
_W_PER_B = _NW // SB
_K_PER_W = K_SLOTS // _W_PER_B


def _sc_gather_body(idx_hbm, table_hbm, out_hbm, idx_v, rows_v, sem):
    wid = lax.axis_index("s") * _NC + lax.axis_index("c")
    b = wid // _W_PER_B
    col = (wid % _W_PER_B) * _K_PER_W
    pltpu.sync_copy(idx_hbm.at[FB + b, pl.ds(col, _K_PER_W)], idx_v)
    pltpu.async_copy(table_hbm.at[idx_v], rows_v, sem).wait()
    pltpu.sync_copy(rows_v, out_hbm.at[b, pl.ds(col, _K_PER_W)])


_sc_gather_half = pl.kernel(
    _sc_gather_body,
    out_type=jax.ShapeDtypeStruct((SB, K_SLOTS, DIM), jnp.float32),
    mesh=plsc.VectorSubcoreMesh(core_axis_name="c", subcore_axis_name="s"),
    scratch_types=[
        pltpu.VMEM((_K_PER_W,), jnp.int32),
        pltpu.VMEM((_K_PER_W, DIM), jnp.float32),
        pltpu.SemaphoreType.DMA,
    ],
)

_MM_DIMS = (((1,), (1,)), ((0,), (0,)))


def _mm_fused_body(idx_ref, attn_ref, table_ref, out_ref, rows_scr, sems):
    g = pl.program_id(0)
    n_steps = FB // BB

    def issue(gi, slot):
        for j in range(BB):
            for k in range(K_SLOTS):
                pltpu.make_async_copy(
                    table_ref.at[idx_ref[gi * BB + j, k]],
                    rows_scr.at[slot, j, k],
                    sems.at[slot],
                ).start()

    def drain(gi, slot):
        for j in range(BB):
            for k in range(K_SLOTS):
                pltpu.make_async_copy(
                    table_ref.at[idx_ref[gi * BB + j, k]],
                    rows_scr.at[slot, j, k],
                    sems.at[slot],
                ).wait()

    @pl.when(g == 0)
    def _():
        issue(0, 0)

    @pl.when(g + 1 < n_steps)
    def _():
        issue(g + 1, (g + 1) % 2)

    drain(g, g % 2)
    out_ref[...] = lax.dot_general(
        attn_ref[...],
        rows_scr[g % 2],
        _MM_DIMS,
        preferred_element_type=jnp.float32,
    )


def _mm_body1(attn_ref, rows_ref, prev_ref, out_ref):
    del prev_ref
    out_ref[...] = lax.dot_general(
        attn_ref[...],
        rows_ref[...],
        _MM_DIMS,
        preferred_element_type=jnp.float32,
    )


@jax.jit
def kernel(indices, attn_map, prompt):
    rows1 = _sc_gather_half(indices, prompt)
    out0 = pl.pallas_call(
        _mm_fused_body,
        grid=(FB // BB,),
        in_specs=[
            pl.BlockSpec(memory_space=pltpu.SMEM),
            pl.BlockSpec((BB, K_SLOTS, N), lambda g: (g, 0, 0)),
            pl.BlockSpec(memory_space=pl.ANY),
        ],
        out_specs=pl.BlockSpec((BB, N, DIM), lambda g: (g, 0, 0)),
        out_shape=jax.ShapeDtypeStruct((B, N, DIM), jnp.float32),
        scratch_shapes=[
            pltpu.VMEM((2, BB, K_SLOTS, DIM), jnp.float32),
            pltpu.SemaphoreType.DMA((2,)),
        ],
    )(indices, attn_map, prompt)
    out = pl.pallas_call(
        _mm_body1,
        grid=(SB // BB,),
        in_specs=[
            pl.BlockSpec((BB, K_SLOTS, N), lambda g: (g + FB // BB, 0, 0)),
            pl.BlockSpec((BB, K_SLOTS, DIM), lambda g: (g, 0, 0)),
            pl.BlockSpec(memory_space=pl.ANY),
        ],
        out_specs=pl.BlockSpec((BB, N, DIM), lambda g: (g + FB // BB, 0, 0)),
        out_shape=jax.ShapeDtypeStruct((B, N, DIM), jnp.float32),
        input_output_aliases={2: 0},
    )(attn_map, rows1, out0)
    return out
